# Initial kernel scaffold; baseline (speedup 1.0000x reference)
import jax
import jax.numpy as jnp
from jax.experimental import pallas as pl


def kernel(p, x, params):
    raise NotImplementedError("write your pallas kernel here")


# speedup vs baseline: 10.5624x; 10.5624x over previous
import functools

import jax
import jax.numpy as jnp
from jax import lax
from jax.experimental import pallas as pl
from jax.experimental.pallas import tpu as pltpu
from jax.experimental.pallas import tpu_sc as plsc

C = 64
K = 16
PAD = 16
EPS = 1e-5


def _qkv_body(xt_ref, wq_ref, wk_ref, wv_ref, bias_ref, q_ref, k_ref, v_ref):
    xt = xt_ref[...]
    b = bias_ref[...]
    q_ref[...] = jnp.dot(xt, wq_ref[...], preferred_element_type=jnp.float32) + b[0:1, :]
    k_ref[...] = jnp.dot(xt, wk_ref[...], preferred_element_type=jnp.float32) + b[1:2, :]
    v_ref[...] = jnp.dot(xt, wv_ref[...], preferred_element_type=jnp.float32) + b[2:3, :]


def _make_qkv(xt, wqT, wkT, wvT, bias3):
    rows = xt.shape[0]
    tr = 512
    grid = (rows // tr,)
    spec_rows = pl.BlockSpec((tr, C), lambda i: (i, 0))
    spec_w = pl.BlockSpec((C, C), lambda i: (0, 0))
    spec_b = pl.BlockSpec((8, C), lambda i: (0, 0))
    out = pl.pallas_call(
        _qkv_body,
        grid=grid,
        in_specs=[spec_rows, spec_w, spec_w, spec_w, spec_b],
        out_specs=[spec_rows, spec_rows, spec_rows],
        out_shape=[jax.ShapeDtypeStruct((rows, C), jnp.float32)] * 3,
    )(xt, wqT, wkT, wvT, bias3)
    return out


def _knn_body(pr_ref, pa_ref, idx_ref, rel_ref):
    n = pa_ref.shape[2]
    r = pr_ref.shape[1]
    b = pl.program_id(0)
    pr = pr_ref[0]
    pa = pa_ref[0]
    sqr = jnp.sum(pr * pr, axis=1, keepdims=True)
    sqa = jnp.sum(pa * pa, axis=0, keepdims=True)
    dots = lax.dot_general(pr, pa, (((1,), (0,)), ((), ())),
                           preferred_element_type=jnp.float32)
    d2 = jnp.maximum(sqr + sqa - 2.0 * dots, 0.0)
    valid = jnp.any(pa != 0.0, axis=0, keepdims=True)
    inf = jnp.float32(jnp.inf)
    d2 = jnp.where(valid, d2, inf)
    col = lax.broadcasted_iota(jnp.int32, (r, n), 1)
    lane = lax.broadcasted_iota(jnp.int32, (r, K), 1)
    idx_tile = jnp.zeros((r, K), jnp.int32)
    big = jnp.int32(2 ** 30)
    for k in range(K):
        m = jnp.min(d2, axis=1, keepdims=True)
        hit = d2 <= m
        idxv = jnp.min(jnp.where(hit, col, big), axis=1, keepdims=True)
        idx_tile = jnp.where(lane == k, idxv, idx_tile)
        sel = (col == idxv)
        d2 = jnp.where(sel, inf, d2)
        wsel = sel.astype(jnp.float32)
        p_sel = lax.dot_general(wsel, pa, (((1,), (1,)), ((), ())),
                                preferred_element_type=jnp.float32)
        rel_ref[0, 0, k] = pr - p_sel
    idx_ref[0] = idx_tile + b * n


def _make_knn(p_rows, p_cols):
    bsz, n, _ = p_rows.shape
    r = 256
    grid = (bsz, n // r)
    idx = pl.pallas_call(
        _knn_body,
        grid=grid,
        in_specs=[
            pl.BlockSpec((1, r, PAD), lambda b, t: (b, t, 0)),
            pl.BlockSpec((1, PAD, n), lambda b, t: (b, 0, 0)),
        ],
        out_specs=[
            pl.BlockSpec((1, r, K), lambda b, t: (b, t, 0)),
            pl.BlockSpec((1, 1, K, r, PAD), lambda b, t: (b, t, 0, 0, 0)),
        ],
        out_shape=[
            jax.ShapeDtypeStruct((bsz, n, K), jnp.int32),
            jax.ShapeDtypeStruct((bsz, n // r, K, r, PAD), jnp.float32),
        ],
    )(p_rows, p_cols)
    return idx


def _make_gather(idx2d, kvtab, rows):
    nw = 32
    ch = 128
    rpw = rows // nw
    nch = rpw // ch
    w = kvtab.shape[1]
    mesh = plsc.VectorSubcoreMesh(core_axis_name="c", subcore_axis_name="s")

    @functools.partial(
        pl.kernel,
        mesh=mesh,
        out_type=jax.ShapeDtypeStruct((rows, w), jnp.float32),
        scratch_types=[
            pltpu.VMEM((nch, ch), jnp.int32),
            pltpu.VMEM((ch, w), jnp.float32),
            pltpu.SemaphoreType.DMA,
        ],
    )
    def gath(idx_hbm, kv_hbm, nkv_hbm, idx_v, buf, sem):
        wid = lax.axis_index("s") * 2 + lax.axis_index("c")
        base = wid * rpw
        pltpu.sync_copy(idx_hbm.at[pl.ds(wid * nch, nch)], idx_v)

        def body(i, carry):
            pltpu.async_copy(kv_hbm.at[idx_v.at[i]], buf, sem).wait()
            pltpu.sync_copy(buf, nkv_hbm.at[pl.ds(base + i * ch, ch)])
            return carry

        lax.fori_loop(0, nch, body, 0)

    return gath(idx2d, kvtab)


def _c1_body(rel_ref, g_ref):
    step = pl.program_id(0)
    rel = rel_ref[...]
    lane = lax.broadcasted_iota(jnp.int32, rel.shape, 1)
    ones_col = jnp.where(lane == 0, 1.0, 0.0)
    aug = jnp.concatenate([rel, ones_col.astype(jnp.float32)], axis=1)
    g = lax.dot_general(aug, aug, (((0,), (0,)), ((), ())),
                        preferred_element_type=jnp.float32)

    @pl.when(step == 0)
    def _():
        g_ref[...] = g

    @pl.when(step != 0)
    def _():
        g_ref[...] = g_ref[...] + g


def _make_c1(rel_g):
    rows = rel_g.shape[0]
    tr = 2048
    grid = (rows // tr,)
    g = pl.pallas_call(
        _c1_body,
        grid=grid,
        in_specs=[pl.BlockSpec((tr, PAD), lambda i: (i, 0))],
        out_specs=pl.BlockSpec((2 * PAD, 2 * PAD), lambda i: (0, 0)),
        out_shape=jax.ShapeDtypeStruct((2 * PAD, 2 * PAD), jnp.float32),
    )(rel_g)
    return g


def _c2_body(nkv_ref, rel_ref, q_ref, sexp_ref,
             w1_ref, w2_ref, cf_ref, pre_ref, nvr_ref, st_ref):
    step = pl.program_id(0)
    sexp = sexp_ref[...]
    cf = cf_ref[...]
    nkv = nkv_ref[...]
    h = jnp.dot(rel_ref[...], w1_ref[...], preferred_element_type=jnp.float32) + cf[0:1, :]
    h = jnp.maximum(h, 0.0)
    nr = jnp.dot(h, w2_ref[...], preferred_element_type=jnp.float32) + cf[1:2, :]
    qx = jnp.dot(sexp, q_ref[...], preferred_element_type=jnp.float32)
    pre = qx - nkv[:, :C] + nr
    pre_ref[...] = pre
    nvr_ref[...] = nkv[:, C:] + nr
    s = jnp.sum(pre, axis=0, keepdims=True)
    ss = jnp.sum(pre * pre, axis=0, keepdims=True)
    st = jnp.concatenate([s, ss, jnp.zeros((6, s.shape[1]), jnp.float32)], axis=0)

    @pl.when(step == 0)
    def _():
        st_ref[...] = st

    @pl.when(step != 0)
    def _():
        st_ref[...] = st_ref[...] + st


def _make_c2(nkv, rel_g, qT, sexp, w1s, w2T, coefs):
    rows = nkv.shape[0]
    tr = 2048
    pt = tr // K
    grid = (rows // tr,)
    spec_rows = pl.BlockSpec((tr, C), lambda i: (i, 0))
    pre, nvr, st = pl.pallas_call(
        _c2_body,
        grid=grid,
        in_specs=[
            pl.BlockSpec((tr, 2 * C), lambda i: (i, 0)),
            pl.BlockSpec((tr, PAD), lambda i: (i, 0)),
            pl.BlockSpec((pt, C), lambda i: (i, 0)),
            pl.BlockSpec((tr, pt), lambda i: (0, 0)),
            pl.BlockSpec((PAD, C), lambda i: (0, 0)),
            pl.BlockSpec((C, C), lambda i: (0, 0)),
            pl.BlockSpec((8, C), lambda i: (0, 0)),
        ],
        out_specs=[spec_rows, spec_rows, pl.BlockSpec((8, C), lambda i: (0, 0))],
        out_shape=[
            jax.ShapeDtypeStruct((rows, C), jnp.float32),
            jax.ShapeDtypeStruct((rows, C), jnp.float32),
            jax.ShapeDtypeStruct((8, C), jnp.float32),
        ],
    )(nkv, rel_g, qT, sexp, w1s, w2T, coefs)
    return pre, nvr, st


def _c3_body(pre_ref, wa1_ref, cf_ref, a1_ref, st_ref):
    step = pl.program_id(0)
    cf = cf_ref[...]
    a = jnp.maximum(pre_ref[...] * cf[2:3, :] + cf[3:4, :], 0.0)
    a1 = jnp.dot(a, wa1_ref[...], preferred_element_type=jnp.float32)
    a1_ref[...] = a1
    s = jnp.sum(a1, axis=0, keepdims=True)
    ss = jnp.sum(a1 * a1, axis=0, keepdims=True)
    st = jnp.concatenate([s, ss, jnp.zeros((6, s.shape[1]), jnp.float32)], axis=0)

    @pl.when(step == 0)
    def _():
        st_ref[...] = st

    @pl.when(step != 0)
    def _():
        st_ref[...] = st_ref[...] + st


def _make_c3(pre, wa1T, coefs):
    rows = pre.shape[0]
    tr = 2048
    grid = (rows // tr,)
    spec_rows = pl.BlockSpec((tr, C), lambda i: (i, 0))
    a1, st = pl.pallas_call(
        _c3_body,
        grid=grid,
        in_specs=[
            spec_rows,
            pl.BlockSpec((C, C), lambda i: (0, 0)),
            pl.BlockSpec((8, C), lambda i: (0, 0)),
        ],
        out_specs=[spec_rows, pl.BlockSpec((8, C), lambda i: (0, 0))],
        out_shape=[
            jax.ShapeDtypeStruct((rows, C), jnp.float32),
            jax.ShapeDtypeStruct((8, C), jnp.float32),
        ],
    )(pre, wa1T, coefs)
    return a1, st


def _c4_body(a1_ref, nvr_ref, sexp_ref, wa2_ref, cf_ref, y_ref):
    cf = cf_ref[...]
    sexp = sexp_ref[...]
    a = jnp.maximum(a1_ref[...] * cf[4:5, :] + cf[5:6, :], 0.0)
    t = jnp.dot(a, wa2_ref[...], preferred_element_type=jnp.float32) + cf[6:7, :]
    e = jnp.exp(t)
    den = lax.dot_general(sexp, e, (((0,), (0,)), ((), ())),
                          preferred_element_type=jnp.float32)
    den_x = jnp.dot(sexp, den, preferred_element_type=jnp.float32)
    attn = e / den_x
    contrib = attn * nvr_ref[...]
    y_ref[...] = lax.dot_general(sexp, contrib, (((0,), (0,)), ((), ())),
                                 preferred_element_type=jnp.float32)


def _make_c4(a1, nvr, sexp, wa2T, coefs):
    rows = a1.shape[0]
    tr = 2048
    pt = tr // K
    grid = (rows // tr,)
    spec_rows = pl.BlockSpec((tr, C), lambda i: (i, 0))
    y = pl.pallas_call(
        _c4_body,
        grid=grid,
        in_specs=[
            spec_rows,
            spec_rows,
            pl.BlockSpec((tr, pt), lambda i: (0, 0)),
            pl.BlockSpec((C, C), lambda i: (0, 0)),
            pl.BlockSpec((8, C), lambda i: (0, 0)),
        ],
        out_specs=pl.BlockSpec((pt, C), lambda i: (i, 0)),
        out_shape=jax.ShapeDtypeStruct((rows // K, C), jnp.float32),
    )(a1, nvr, sexp, wa2T, coefs)
    return y


def kernel(p, x, params):
    bsz, n, _ = p.shape
    rows = bsz * n * K
    npts = bsz * n

    xt = jnp.transpose(x, (0, 2, 1)).reshape(npts, C)
    p_pad = jnp.pad(p, ((0, 0), (0, 0), (0, PAD - 3)))
    p_cols = jnp.transpose(p_pad, (0, 2, 1))
    p_rows_flat = p_pad.reshape(npts, PAD)

    bias3 = jnp.zeros((8, C), jnp.float32)
    bias3 = bias3.at[0].set(params['bq']).at[1].set(params['bk']).at[2].set(params['bv'])
    qT, kT, vT = _make_qkv(xt, params['Wq'].T, params['Wk'].T, params['Wv'].T, bias3)

    idx, rel5 = _make_knn(p_pad, p_cols)
    idx2d = idx.reshape(rows // 128, 128)
    rel_g = jnp.transpose(rel5, (0, 1, 3, 2, 4)).reshape(rows, PAD)

    kvtab = jnp.concatenate([kT, vT], axis=1)
    nkv = _make_gather(idx2d, kvtab, rows)

    tr = 2048
    pt = tr // K
    sexp = (lax.broadcasted_iota(jnp.int32, (tr, pt), 0) // K ==
            lax.broadcasted_iota(jnp.int32, (tr, pt), 1)).astype(jnp.float32)

    g = _make_c1(rel_g)
    m = jnp.float32(rows)
    srel = g[PAD, :PAD]
    g2 = g[:PAD, :PAD]
    w1pad = jnp.pad(params['Wpe1'], ((0, 0), (0, PAD - 3)))
    mu1 = w1pad @ (srel / m)
    e2 = jnp.einsum('ck,kl,cl->c', w1pad, g2 / m, w1pad)
    var1 = e2 - mu1 * mu1
    scale1 = params['g_pe'] * lax.rsqrt(var1 + EPS)
    shift1 = params['b_pe'] - mu1 * scale1
    w1s = (w1pad * scale1[:, None]).T

    coefs = jnp.zeros((8, C), jnp.float32)
    coefs = coefs.at[0].set(shift1).at[1].set(params['bpe2']).at[6].set(params['ba2'])

    pre, nvr, st2 = _make_c2(nkv, rel_g, qT, sexp, w1s, params['Wpe2'].T, coefs)
    mu2 = st2[0] / m
    var2 = st2[1] / m - mu2 * mu2
    scale2 = params['g_a1'] * lax.rsqrt(var2 + EPS)
    shift2 = params['b_a1'] - mu2 * scale2
    coefs = coefs.at[2].set(scale2).at[3].set(shift2)

    a1, st3 = _make_c3(pre, params['Wa1'].T, coefs)
    mu3 = st3[0] / m
    var3 = st3[1] / m - mu3 * mu3
    scale3 = params['g_a2'] * lax.rsqrt(var3 + EPS)
    shift3 = params['b_a2'] - mu3 * scale3
    coefs = coefs.at[4].set(scale3).at[5].set(shift3)

    y = _make_c4(a1, nvr, sexp, params['Wa2'].T, coefs)
    return jnp.transpose(y.reshape(bsz, n, C), (0, 2, 1))
